# baseline (speedup 1.0000x reference)
import functools

import jax
import jax.numpy as jnp
from jax import lax
from jax.experimental import pallas as pl
from jax.experimental.pallas import tpu as pltpu
from jax.experimental.pallas import tpu_sc as plsc

_NUM_NETS = 50000
_NUM_PINS = 200000
_NBX = 256
_NBY = 256
_XL = 0.0
_YL = 0.0
_BSX = 1.0 / _NBX
_BSY = 1.0 / _NBY
_UNIT_H_CAP = 1.5625
_UNIT_V_CAP = 1.45

_NW = 32
_CHUNK = 128
_KCH = 49
_PPW = _KCH * _CHUNK
_PAD_PINS = _NW * _PPW
_PIN_ROWS = _PAD_PINS // _CHUNK
_PIN_ROWS_PAD = 1576

_NB = 2048
_SUB = 64
_GRID = (_NUM_NETS + _NB - 1) // _NB
_PAD_NETS = _GRID * _NB


def _sc_gather_body(x_hbm, y_hbm, fnp_hbm, px_hbm, py_hbm, idx_v, xv, yv, semx, semy):
    wid = lax.axis_index("s") * 2 + lax.axis_index("c")
    base = wid * _PPW
    pltpu.sync_copy(fnp_hbm.at[pl.ds(base, _PPW)], idx_v)

    def fire(j, _):
        o = j * _CHUNK
        pltpu.async_copy(x_hbm.at[idx_v.at[pl.ds(o, _CHUNK)]], xv.at[pl.ds(o, _CHUNK)], semx)
        pltpu.async_copy(y_hbm.at[idx_v.at[pl.ds(o, _CHUNK)]], yv.at[pl.ds(o, _CHUNK)], semy)
        return 0

    lax.fori_loop(0, _KCH, fire, 0)

    def drain(j, _):
        o = j * _CHUNK
        pltpu.make_async_copy(x_hbm.at[idx_v.at[pl.ds(o, _CHUNK)]], xv.at[pl.ds(o, _CHUNK)], semx).wait()
        pltpu.make_async_copy(y_hbm.at[idx_v.at[pl.ds(o, _CHUNK)]], yv.at[pl.ds(o, _CHUNK)], semy).wait()
        return 0

    lax.fori_loop(0, _KCH, drain, 0)
    pltpu.sync_copy(xv, px_hbm.at[pl.ds(base, _PPW)])
    pltpu.sync_copy(yv, py_hbm.at[pl.ds(base, _PPW)])


@functools.cache
def _sc_gather():
    return pl.kernel(
        _sc_gather_body,
        out_type=(
            jax.ShapeDtypeStruct((_PAD_PINS,), jnp.float32),
            jax.ShapeDtypeStruct((_PAD_PINS,), jnp.float32),
        ),
        mesh=plsc.VectorSubcoreMesh(core_axis_name="c", subcore_axis_name="s"),
        scratch_types=[
            pltpu.VMEM((_PPW,), jnp.int32),
            pltpu.VMEM((_PPW,), jnp.float32),
            pltpu.VMEM((_PPW,), jnp.float32),
            pltpu.SemaphoreType.DMA,
            pltpu.SemaphoreType.DMA,
        ],
    )


def _tc_body(starts_s, ends_s, starts_v, ends_v, w_v, px_v, py_v, out_ref, acc):
    i = pl.program_id(0)
    big = jnp.float32(3.0e38)
    lane = lax.broadcasted_iota(jnp.int32, (1, _CHUNK), 1)

    parts = []
    for s in range(_NB // _SUB):
        s_a = starts_s[0, 0, s * _SUB]
        s_b = ends_s[0, 0, s * _SUB + _SUB - 1]
        r0 = lax.div(s_a, _CHUNK)
        r1 = lax.div(s_b - 1, _CHUNK)
        nch = jnp.maximum(r1 - r0 + 1, 0)
        st = starts_v[0, s * _SUB:(s + 1) * _SUB].astype(jnp.int32)
        en = ends_v[0, s * _SUB:(s + 1) * _SUB].astype(jnp.int32)

        last = jnp.maximum(nch - 1, 0)

        def chunk(c, carry, st=st, en=en, r0=r0, last=last):
            cxmn, cxmx, cymn, cymx = carry
            for j in range(8):
                row = r0 + jnp.minimum(8 * c + j, last)
                pid = lane + row * _CHUNK
                pxc = px_v[pl.ds(row, 1), :]
                pyc = py_v[pl.ds(row, 1), :]
                member = (pid >= st) & (pid < en)
                cxmn = jnp.minimum(cxmn, jnp.where(member, pxc, big))
                cxmx = jnp.maximum(cxmx, jnp.where(member, pxc, -big))
                cymn = jnp.minimum(cymn, jnp.where(member, pyc, big))
                cymx = jnp.maximum(cymx, jnp.where(member, pyc, -big))
            return cxmn, cxmx, cymn, cymx

        cfull = jnp.full((_SUB, _CHUNK), big, jnp.float32)
        cxmn, cxmx, cymn, cymx = lax.fori_loop(
            0, lax.div(nch + 7, 8), chunk, (cfull, -cfull, cfull, -cfull))
        parts.append((jnp.min(cxmn, axis=1, keepdims=True),
                      jnp.max(cxmx, axis=1, keepdims=True),
                      jnp.min(cymn, axis=1, keepdims=True),
                      jnp.max(cymx, axis=1, keepdims=True)))

    xmn = jnp.concatenate([p[0] for p in parts], axis=0)
    xmx = jnp.concatenate([p[1] for p in parts], axis=0)
    ymn = jnp.concatenate([p[2] for p in parts], axis=0)
    ymx = jnp.concatenate([p[3] for p in parts], axis=0)
    st = starts_v[0].astype(jnp.int32)
    en = ends_v[0].astype(jnp.int32)

    valid = en > st
    xmn = jnp.where(valid, xmn, _XL)
    xmx = jnp.where(valid, xmx, _XL)
    ymn = jnp.where(valid, ymn, _YL)
    ymx = jnp.where(valid, ymx, _YL)
    wt = jnp.where(valid, w_v[0], 0.0)

    eps = jnp.float32(jnp.finfo(jnp.float32).eps)
    bxl = lax.broadcasted_iota(jnp.int32, (1, _NBX), 1).astype(jnp.float32) * _BSX + _XL
    byl = lax.broadcasted_iota(jnp.int32, (1, _NBY), 1).astype(jnp.float32) * _BSY + _YL
    ox = jnp.maximum(jnp.minimum(xmx, bxl + _BSX) - jnp.maximum(xmn, bxl), 0.0)
    oy = jnp.maximum(jnp.minimum(ymx, byl + _BSY) - jnp.maximum(ymn, byl), 0.0)
    wt_h = wt / (ymx - ymn + eps)
    wt_v = wt / (xmx - xmn + eps)
    a2 = jnp.concatenate([ox * wt_h, ox * wt_v], axis=1)

    contrib = lax.dot_general(a2, oy, (((0,), (0,)), ((), ())),
                              preferred_element_type=jnp.float32)

    @pl.when(i == 0)
    def _():
        acc[...] = jnp.zeros((2 * _NBX, _NBY), jnp.float32)

    acc[...] += contrib

    @pl.when(i == pl.num_programs(0) - 1)
    def _():
        bin_area = _BSX * _BSY
        sh = jnp.float32(1.0 / (bin_area * _UNIT_H_CAP))
        sv = jnp.float32(1.0 / (bin_area * _UNIT_V_CAP))
        out_ref[...] = jnp.maximum(jnp.abs(acc[0:_NBX, :]) * sh,
                                   jnp.abs(acc[_NBX:, :]) * sv)


def _clampi(i):
    return (jnp.minimum(i, _GRID - 1), 0, 0)


def _tc_rudy(starts3, ends3, startsv3, endsv3, w3, px2, py2):
    return pl.pallas_call(
        _tc_body,
        grid=(_GRID,),
        in_specs=[
            pl.BlockSpec((1, 1, _NB), _clampi, memory_space=pltpu.SMEM),
            pl.BlockSpec((1, 1, _NB), _clampi, memory_space=pltpu.SMEM),
            pl.BlockSpec((1, _NB, 1), _clampi),
            pl.BlockSpec((1, _NB, 1), _clampi),
            pl.BlockSpec((1, _NB, 1), _clampi),
            pl.BlockSpec((_PIN_ROWS_PAD, _CHUNK), lambda i: (0, 0)),
            pl.BlockSpec((_PIN_ROWS_PAD, _CHUNK), lambda i: (0, 0)),
        ],
        out_specs=pl.BlockSpec((_NBX, _NBY), lambda i: (0, 0)),
        out_shape=jax.ShapeDtypeStruct((_NBX, _NBY), jnp.float32),
        scratch_shapes=[pltpu.VMEM((2 * _NBX, _NBY), jnp.float32)],
    )(starts3, ends3, startsv3, endsv3, w3, px2, py2)


def kernel(pin_pos, netpin_start, flat_netpin, net_weights):
    x = pin_pos[:_NUM_PINS]
    y = pin_pos[_NUM_PINS:]
    fnp1 = jnp.pad(flat_netpin, (0, _PAD_PINS - _NUM_PINS))
    px1, py1 = _sc_gather()(x, y, fnp1)
    px2 = jnp.pad(px1.reshape(_PIN_ROWS, _CHUNK),
                  ((0, _PIN_ROWS_PAD - _PIN_ROWS), (0, 0)))
    py2 = jnp.pad(py1.reshape(_PIN_ROWS, _CHUNK),
                  ((0, _PIN_ROWS_PAD - _PIN_ROWS), (0, 0)))

    starts = jnp.pad(netpin_start[:-1], (0, _PAD_NETS - _NUM_NETS),
                     constant_values=_NUM_PINS)
    ends = jnp.pad(netpin_start[1:], (0, _PAD_NETS - _NUM_NETS),
                   constant_values=_NUM_PINS)
    wpad = jnp.pad(net_weights, (0, _PAD_NETS - _NUM_NETS))
    starts3 = starts.reshape(_GRID, 1, _NB)
    ends3 = ends.reshape(_GRID, 1, _NB)
    startsv3 = starts.reshape(_GRID, _NB, 1)
    endsv3 = ends.reshape(_GRID, _NB, 1)
    w3 = wpad.reshape(_GRID, _NB, 1)
    return _tc_rudy(starts3, ends3, startsv3, endsv3, w3, px2, py2)


# speedup vs baseline: 1.0618x; 1.0618x over previous
import functools

import jax
import jax.numpy as jnp
from jax import lax
from jax.experimental import pallas as pl
from jax.experimental.pallas import tpu as pltpu
from jax.experimental.pallas import tpu_sc as plsc

_NUM_NETS = 50000
_NUM_PINS = 200000
_NBX = 256
_NBY = 256
_XL = 0.0
_YL = 0.0
_BSX = 1.0 / _NBX
_BSY = 1.0 / _NBY
_UNIT_H_CAP = 1.5625
_UNIT_V_CAP = 1.45

_NW = 32
_CHUNK = 128
_KCH = 49
_PPW = _KCH * _CHUNK
_PAD_PINS = _NW * _PPW
_PIN_ROWS = _PAD_PINS // _CHUNK
_PIN_ROWS_PAD = 1576

_NB = 4096
_SUB = 64
_GRID = (_NUM_NETS + _NB - 1) // _NB
_PAD_NETS = _GRID * _NB


def _sc_gather_body(x_hbm, y_hbm, fnp_hbm, px_hbm, py_hbm, idx_v, xv, yv, semx, semy):
    wid = lax.axis_index("s") * 2 + lax.axis_index("c")
    base = wid * _PPW
    pltpu.sync_copy(fnp_hbm.at[pl.ds(base, _PPW)], idx_v)

    def fire(j, _):
        o = j * _CHUNK
        pltpu.async_copy(x_hbm.at[idx_v.at[pl.ds(o, _CHUNK)]], xv.at[pl.ds(o, _CHUNK)], semx)
        pltpu.async_copy(y_hbm.at[idx_v.at[pl.ds(o, _CHUNK)]], yv.at[pl.ds(o, _CHUNK)], semy)
        return 0

    lax.fori_loop(0, _KCH, fire, 0)

    def drain(j, _):
        o = j * _CHUNK
        pltpu.make_async_copy(x_hbm.at[idx_v.at[pl.ds(o, _CHUNK)]], xv.at[pl.ds(o, _CHUNK)], semx).wait()
        pltpu.make_async_copy(y_hbm.at[idx_v.at[pl.ds(o, _CHUNK)]], yv.at[pl.ds(o, _CHUNK)], semy).wait()
        return 0

    lax.fori_loop(0, _KCH, drain, 0)
    pltpu.sync_copy(xv, px_hbm.at[pl.ds(base, _PPW)])
    pltpu.sync_copy(yv, py_hbm.at[pl.ds(base, _PPW)])


@functools.cache
def _sc_gather():
    return pl.kernel(
        _sc_gather_body,
        out_type=(
            jax.ShapeDtypeStruct((_PAD_PINS,), jnp.float32),
            jax.ShapeDtypeStruct((_PAD_PINS,), jnp.float32),
        ),
        mesh=plsc.VectorSubcoreMesh(core_axis_name="c", subcore_axis_name="s"),
        scratch_types=[
            pltpu.VMEM((_PPW,), jnp.int32),
            pltpu.VMEM((_PPW,), jnp.float32),
            pltpu.VMEM((_PPW,), jnp.float32),
            pltpu.SemaphoreType.DMA,
            pltpu.SemaphoreType.DMA,
        ],
    )


def _tc_body(starts_s, ends_s, starts_v, ends_v, w_v, px_v, py_v, out_ref, acc):
    i = pl.program_id(0)
    big = jnp.float32(3.0e38)
    lane = lax.broadcasted_iota(jnp.int32, (1, _CHUNK), 1)

    parts = []
    for s in range(_NB // _SUB):
        s_a = starts_s[0, 0, s * _SUB]
        s_b = ends_s[0, 0, s * _SUB + _SUB - 1]
        r0 = lax.div(s_a, _CHUNK)
        r1 = lax.div(s_b - 1, _CHUNK)
        nch = jnp.maximum(r1 - r0 + 1, 0)
        st = starts_v[0, s * _SUB:(s + 1) * _SUB].astype(jnp.int32)
        en = ends_v[0, s * _SUB:(s + 1) * _SUB].astype(jnp.int32)

        last = jnp.maximum(nch - 1, 0)

        def chunk(c, carry, st=st, en=en, r0=r0, last=last):
            cxmn, cxmx, cymn, cymx = carry
            for j in range(4):
                row = r0 + jnp.minimum(4 * c + j, last)
                pid = lane + row * _CHUNK
                pxc = px_v[pl.ds(row, 1), :]
                pyc = py_v[pl.ds(row, 1), :]
                member = (pid >= st) & (pid < en)
                cxmn = jnp.minimum(cxmn, jnp.where(member, pxc, big))
                cxmx = jnp.maximum(cxmx, jnp.where(member, pxc, -big))
                cymn = jnp.minimum(cymn, jnp.where(member, pyc, big))
                cymx = jnp.maximum(cymx, jnp.where(member, pyc, -big))
            return cxmn, cxmx, cymn, cymx

        cfull = jnp.full((_SUB, _CHUNK), big, jnp.float32)
        cxmn, cxmx, cymn, cymx = lax.fori_loop(
            0, lax.div(nch + 3, 4), chunk, (cfull, -cfull, cfull, -cfull))
        parts.append((jnp.min(cxmn, axis=1, keepdims=True),
                      jnp.max(cxmx, axis=1, keepdims=True),
                      jnp.min(cymn, axis=1, keepdims=True),
                      jnp.max(cymx, axis=1, keepdims=True)))

    xmn = jnp.concatenate([p[0] for p in parts], axis=0)
    xmx = jnp.concatenate([p[1] for p in parts], axis=0)
    ymn = jnp.concatenate([p[2] for p in parts], axis=0)
    ymx = jnp.concatenate([p[3] for p in parts], axis=0)
    st = starts_v[0].astype(jnp.int32)
    en = ends_v[0].astype(jnp.int32)

    valid = en > st
    xmn = jnp.where(valid, xmn, _XL)
    xmx = jnp.where(valid, xmx, _XL)
    ymn = jnp.where(valid, ymn, _YL)
    ymx = jnp.where(valid, ymx, _YL)
    wt = jnp.where(valid, w_v[0], 0.0)

    eps = jnp.float32(jnp.finfo(jnp.float32).eps)
    bxl = lax.broadcasted_iota(jnp.int32, (1, _NBX), 1).astype(jnp.float32) * _BSX + _XL
    byl = lax.broadcasted_iota(jnp.int32, (1, _NBY), 1).astype(jnp.float32) * _BSY + _YL
    ox = jnp.maximum(jnp.minimum(xmx, bxl + _BSX) - jnp.maximum(xmn, bxl), 0.0)
    oy = jnp.maximum(jnp.minimum(ymx, byl + _BSY) - jnp.maximum(ymn, byl), 0.0)
    wt_h = wt / (ymx - ymn + eps)
    wt_v = wt / (xmx - xmn + eps)
    a2 = jnp.concatenate([ox * wt_h, ox * wt_v], axis=1)

    contrib = lax.dot_general(a2, oy, (((0,), (0,)), ((), ())),
                              preferred_element_type=jnp.float32)

    @pl.when(i == 0)
    def _():
        acc[...] = jnp.zeros((2 * _NBX, _NBY), jnp.float32)

    acc[...] += contrib

    @pl.when(i == pl.num_programs(0) - 1)
    def _():
        bin_area = _BSX * _BSY
        sh = jnp.float32(1.0 / (bin_area * _UNIT_H_CAP))
        sv = jnp.float32(1.0 / (bin_area * _UNIT_V_CAP))
        out_ref[...] = jnp.maximum(jnp.abs(acc[0:_NBX, :]) * sh,
                                   jnp.abs(acc[_NBX:, :]) * sv)


def _clampi(i):
    return (jnp.minimum(i, _GRID - 1), 0, 0)


def _tc_rudy(starts3, ends3, startsv3, endsv3, w3, px2, py2):
    return pl.pallas_call(
        _tc_body,
        grid=(_GRID,),
        in_specs=[
            pl.BlockSpec((1, 1, _NB), _clampi, memory_space=pltpu.SMEM),
            pl.BlockSpec((1, 1, _NB), _clampi, memory_space=pltpu.SMEM),
            pl.BlockSpec((1, _NB, 1), _clampi),
            pl.BlockSpec((1, _NB, 1), _clampi),
            pl.BlockSpec((1, _NB, 1), _clampi),
            pl.BlockSpec((_PIN_ROWS_PAD, _CHUNK), lambda i: (0, 0)),
            pl.BlockSpec((_PIN_ROWS_PAD, _CHUNK), lambda i: (0, 0)),
        ],
        out_specs=pl.BlockSpec((_NBX, _NBY), lambda i: (0, 0)),
        out_shape=jax.ShapeDtypeStruct((_NBX, _NBY), jnp.float32),
        scratch_shapes=[pltpu.VMEM((2 * _NBX, _NBY), jnp.float32)],
    )(starts3, ends3, startsv3, endsv3, w3, px2, py2)


def kernel(pin_pos, netpin_start, flat_netpin, net_weights):
    x = pin_pos[:_NUM_PINS]
    y = pin_pos[_NUM_PINS:]
    fnp1 = jnp.pad(flat_netpin, (0, _PAD_PINS - _NUM_PINS))
    px1, py1 = _sc_gather()(x, y, fnp1)
    px2 = jnp.pad(px1.reshape(_PIN_ROWS, _CHUNK),
                  ((0, _PIN_ROWS_PAD - _PIN_ROWS), (0, 0)))
    py2 = jnp.pad(py1.reshape(_PIN_ROWS, _CHUNK),
                  ((0, _PIN_ROWS_PAD - _PIN_ROWS), (0, 0)))

    starts = jnp.pad(netpin_start[:-1], (0, _PAD_NETS - _NUM_NETS),
                     constant_values=_NUM_PINS)
    ends = jnp.pad(netpin_start[1:], (0, _PAD_NETS - _NUM_NETS),
                   constant_values=_NUM_PINS)
    wpad = jnp.pad(net_weights, (0, _PAD_NETS - _NUM_NETS))
    starts3 = starts.reshape(_GRID, 1, _NB)
    ends3 = ends.reshape(_GRID, 1, _NB)
    startsv3 = starts.reshape(_GRID, _NB, 1)
    endsv3 = ends.reshape(_GRID, _NB, 1)
    w3 = wpad.reshape(_GRID, _NB, 1)
    return _tc_rudy(starts3, ends3, startsv3, endsv3, w3, px2, py2)


# speedup vs baseline: 1.0859x; 1.0227x over previous
import functools

import jax
import jax.numpy as jnp
from jax import lax
from jax.experimental import pallas as pl
from jax.experimental.pallas import tpu as pltpu
from jax.experimental.pallas import tpu_sc as plsc

_NUM_NETS = 50000
_NUM_PINS = 200000
_NBX = 256
_NBY = 256
_XL = 0.0
_YL = 0.0
_BSX = 1.0 / _NBX
_BSY = 1.0 / _NBY
_UNIT_H_CAP = 1.5625
_UNIT_V_CAP = 1.45

_NW = 32
_CHUNK = 128
_KCH = 49
_PPW = _KCH * _CHUNK
_PAD_PINS = _NW * _PPW
_PIN_ROWS = _PAD_PINS // _CHUNK
_PIN_ROWS_PAD = 1576

_NB = 2048
_SUB = 64
_GRID = (_NUM_NETS + _NB - 1) // _NB
_PAD_NETS = _GRID * _NB


def _sc_gather_body(x_hbm, y_hbm, fnp_hbm, px_hbm, py_hbm, idx_v, xv, yv, semx, semy):
    wid = lax.axis_index("s") * 2 + lax.axis_index("c")
    base = wid * _PPW
    pltpu.sync_copy(fnp_hbm.at[pl.ds(base, _PPW)], idx_v)

    def fire(j, _):
        o = j * _CHUNK
        pltpu.async_copy(x_hbm.at[idx_v.at[pl.ds(o, _CHUNK)]], xv.at[pl.ds(o, _CHUNK)], semx)
        pltpu.async_copy(y_hbm.at[idx_v.at[pl.ds(o, _CHUNK)]], yv.at[pl.ds(o, _CHUNK)], semy)
        return 0

    lax.fori_loop(0, _KCH, fire, 0)

    def drain(j, _):
        o = j * _CHUNK
        pltpu.make_async_copy(x_hbm.at[idx_v.at[pl.ds(o, _CHUNK)]], xv.at[pl.ds(o, _CHUNK)], semx).wait()
        pltpu.make_async_copy(y_hbm.at[idx_v.at[pl.ds(o, _CHUNK)]], yv.at[pl.ds(o, _CHUNK)], semy).wait()
        return 0

    lax.fori_loop(0, _KCH, drain, 0)
    pltpu.sync_copy(xv, px_hbm.at[pl.ds(base, _PPW)])
    pltpu.sync_copy(yv, py_hbm.at[pl.ds(base, _PPW)])


@functools.cache
def _sc_gather():
    return pl.kernel(
        _sc_gather_body,
        out_type=(
            jax.ShapeDtypeStruct((_PAD_PINS,), jnp.float32),
            jax.ShapeDtypeStruct((_PAD_PINS,), jnp.float32),
        ),
        mesh=plsc.VectorSubcoreMesh(core_axis_name="c", subcore_axis_name="s"),
        scratch_types=[
            pltpu.VMEM((_PPW,), jnp.int32),
            pltpu.VMEM((_PPW,), jnp.float32),
            pltpu.VMEM((_PPW,), jnp.float32),
            pltpu.SemaphoreType.DMA,
            pltpu.SemaphoreType.DMA,
        ],
    )


def _tc_body(starts_s, ends_s, starts_v, ends_v, w_v, px_v, py_v, out_ref, acc):
    i = pl.program_id(0)
    big = jnp.float32(3.0e38)
    lane = lax.broadcasted_iota(jnp.int32, (1, _CHUNK), 1)

    parts = []
    for s in range(_NB // _SUB):
        s_a = starts_s[0, 0, s * _SUB]
        s_b = ends_s[0, 0, s * _SUB + _SUB - 1]
        r0 = lax.div(s_a, _CHUNK)
        r1 = lax.div(s_b - 1, _CHUNK)
        nch = jnp.maximum(r1 - r0 + 1, 0)
        st = starts_v[0, s * _SUB:(s + 1) * _SUB].astype(jnp.int32)
        en = ends_v[0, s * _SUB:(s + 1) * _SUB].astype(jnp.int32)

        last = jnp.maximum(nch - 1, 0)

        def chunk(c, carry, st=st, en=en, r0=r0, last=last):
            cxmn, cxmx, cymn, cymx = carry
            for j in range(3):
                row = r0 + jnp.minimum(3 * c + j, last)
                pid = lane + row * _CHUNK
                pxc = px_v[pl.ds(row, 1), :]
                pyc = py_v[pl.ds(row, 1), :]
                member = (pid >= st) & (pid < en)
                cxmn = jnp.minimum(cxmn, jnp.where(member, pxc, big))
                cxmx = jnp.maximum(cxmx, jnp.where(member, pxc, -big))
                cymn = jnp.minimum(cymn, jnp.where(member, pyc, big))
                cymx = jnp.maximum(cymx, jnp.where(member, pyc, -big))
            return cxmn, cxmx, cymn, cymx

        cfull = jnp.full((_SUB, _CHUNK), big, jnp.float32)
        cxmn, cxmx, cymn, cymx = lax.fori_loop(
            0, lax.div(nch + 2, 3), chunk, (cfull, -cfull, cfull, -cfull))
        parts.append((jnp.min(cxmn, axis=1, keepdims=True),
                      jnp.max(cxmx, axis=1, keepdims=True),
                      jnp.min(cymn, axis=1, keepdims=True),
                      jnp.max(cymx, axis=1, keepdims=True)))

    xmn = jnp.concatenate([p[0] for p in parts], axis=0)
    xmx = jnp.concatenate([p[1] for p in parts], axis=0)
    ymn = jnp.concatenate([p[2] for p in parts], axis=0)
    ymx = jnp.concatenate([p[3] for p in parts], axis=0)
    st = starts_v[0].astype(jnp.int32)
    en = ends_v[0].astype(jnp.int32)

    valid = en > st
    xmn = jnp.where(valid, xmn, _XL)
    xmx = jnp.where(valid, xmx, _XL)
    ymn = jnp.where(valid, ymn, _YL)
    ymx = jnp.where(valid, ymx, _YL)
    wt = jnp.where(valid, w_v[0], 0.0)

    eps = jnp.float32(jnp.finfo(jnp.float32).eps)
    bxl = lax.broadcasted_iota(jnp.int32, (1, _NBX), 1).astype(jnp.float32) * _BSX + _XL
    byl = lax.broadcasted_iota(jnp.int32, (1, _NBY), 1).astype(jnp.float32) * _BSY + _YL
    ox = jnp.maximum(jnp.minimum(xmx, bxl + _BSX) - jnp.maximum(xmn, bxl), 0.0)
    oy = jnp.maximum(jnp.minimum(ymx, byl + _BSY) - jnp.maximum(ymn, byl), 0.0)
    wt_h = wt / (ymx - ymn + eps)
    wt_v = wt / (xmx - xmn + eps)
    a2 = jnp.concatenate([ox * wt_h, ox * wt_v], axis=1)

    contrib = lax.dot_general(a2, oy, (((0,), (0,)), ((), ())),
                              preferred_element_type=jnp.float32)

    @pl.when(i == 0)
    def _():
        acc[...] = jnp.zeros((2 * _NBX, _NBY), jnp.float32)

    acc[...] += contrib

    @pl.when(i == pl.num_programs(0) - 1)
    def _():
        bin_area = _BSX * _BSY
        sh = jnp.float32(1.0 / (bin_area * _UNIT_H_CAP))
        sv = jnp.float32(1.0 / (bin_area * _UNIT_V_CAP))
        out_ref[...] = jnp.maximum(jnp.abs(acc[0:_NBX, :]) * sh,
                                   jnp.abs(acc[_NBX:, :]) * sv)


def _clampi(i):
    return (jnp.minimum(i, _GRID - 1), 0, 0)


def _tc_rudy(starts3, ends3, startsv3, endsv3, w3, px2, py2):
    return pl.pallas_call(
        _tc_body,
        grid=(_GRID,),
        in_specs=[
            pl.BlockSpec((1, 1, _NB), _clampi, memory_space=pltpu.SMEM),
            pl.BlockSpec((1, 1, _NB), _clampi, memory_space=pltpu.SMEM),
            pl.BlockSpec((1, _NB, 1), _clampi),
            pl.BlockSpec((1, _NB, 1), _clampi),
            pl.BlockSpec((1, _NB, 1), _clampi),
            pl.BlockSpec((_PIN_ROWS_PAD, _CHUNK), lambda i: (0, 0)),
            pl.BlockSpec((_PIN_ROWS_PAD, _CHUNK), lambda i: (0, 0)),
        ],
        out_specs=pl.BlockSpec((_NBX, _NBY), lambda i: (0, 0)),
        out_shape=jax.ShapeDtypeStruct((_NBX, _NBY), jnp.float32),
        scratch_shapes=[pltpu.VMEM((2 * _NBX, _NBY), jnp.float32)],
    )(starts3, ends3, startsv3, endsv3, w3, px2, py2)


def kernel(pin_pos, netpin_start, flat_netpin, net_weights):
    x = pin_pos[:_NUM_PINS]
    y = pin_pos[_NUM_PINS:]
    fnp1 = jnp.pad(flat_netpin, (0, _PAD_PINS - _NUM_PINS))
    px1, py1 = _sc_gather()(x, y, fnp1)
    px2 = jnp.pad(px1.reshape(_PIN_ROWS, _CHUNK),
                  ((0, _PIN_ROWS_PAD - _PIN_ROWS), (0, 0)))
    py2 = jnp.pad(py1.reshape(_PIN_ROWS, _CHUNK),
                  ((0, _PIN_ROWS_PAD - _PIN_ROWS), (0, 0)))

    starts = jnp.pad(netpin_start[:-1], (0, _PAD_NETS - _NUM_NETS),
                     constant_values=_NUM_PINS)
    ends = jnp.pad(netpin_start[1:], (0, _PAD_NETS - _NUM_NETS),
                   constant_values=_NUM_PINS)
    wpad = jnp.pad(net_weights, (0, _PAD_NETS - _NUM_NETS))
    starts3 = starts.reshape(_GRID, 1, _NB)
    ends3 = ends.reshape(_GRID, 1, _NB)
    startsv3 = starts.reshape(_GRID, _NB, 1)
    endsv3 = ends.reshape(_GRID, _NB, 1)
    w3 = wpad.reshape(_GRID, _NB, 1)
    return _tc_rudy(starts3, ends3, startsv3, endsv3, w3, px2, py2)


# speedup vs baseline: 1.0913x; 1.0050x over previous
import functools

import jax
import jax.numpy as jnp
from jax import lax
from jax.experimental import pallas as pl
from jax.experimental.pallas import tpu as pltpu
from jax.experimental.pallas import tpu_sc as plsc

_NUM_NETS = 50000
_NUM_PINS = 200000
_NBX = 256
_NBY = 256
_XL = 0.0
_YL = 0.0
_BSX = 1.0 / _NBX
_BSY = 1.0 / _NBY
_UNIT_H_CAP = 1.5625
_UNIT_V_CAP = 1.45

_NW = 32
_CHUNK = 128
_KCH = 49
_PPW = _KCH * _CHUNK
_PAD_PINS = _NW * _PPW
_PIN_ROWS = _PAD_PINS // _CHUNK
_PIN_ROWS_PAD = 1576

_NB = 2048
_SUB = 64
_GRID = (_NUM_NETS + _NB - 1) // _NB
_PAD_NETS = _GRID * _NB


def _sc_gather_body(x_hbm, y_hbm, fnp_hbm, px_hbm, py_hbm, idx_v, xv, yv, semx, semy):
    wid = lax.axis_index("s") * 2 + lax.axis_index("c")
    base = wid * _PPW
    pltpu.sync_copy(fnp_hbm.at[pl.ds(base, _PPW)], idx_v)

    def fire(j, _):
        o = j * _CHUNK
        pltpu.async_copy(x_hbm.at[idx_v.at[pl.ds(o, _CHUNK)]], xv.at[pl.ds(o, _CHUNK)], semx)
        pltpu.async_copy(y_hbm.at[idx_v.at[pl.ds(o, _CHUNK)]], yv.at[pl.ds(o, _CHUNK)], semy)
        return 0

    lax.fori_loop(0, _KCH, fire, 0)

    def drain(j, _):
        o = j * _CHUNK
        pltpu.make_async_copy(x_hbm.at[idx_v.at[pl.ds(o, _CHUNK)]], xv.at[pl.ds(o, _CHUNK)], semx).wait()
        pltpu.make_async_copy(y_hbm.at[idx_v.at[pl.ds(o, _CHUNK)]], yv.at[pl.ds(o, _CHUNK)], semy).wait()
        return 0

    lax.fori_loop(0, _KCH, drain, 0)
    pltpu.sync_copy(xv, px_hbm.at[pl.ds(base, _PPW)])
    pltpu.sync_copy(yv, py_hbm.at[pl.ds(base, _PPW)])


@functools.cache
def _sc_gather():
    return pl.kernel(
        _sc_gather_body,
        out_type=(
            jax.ShapeDtypeStruct((_PAD_PINS,), jnp.float32),
            jax.ShapeDtypeStruct((_PAD_PINS,), jnp.float32),
        ),
        mesh=plsc.VectorSubcoreMesh(core_axis_name="c", subcore_axis_name="s"),
        scratch_types=[
            pltpu.VMEM((_PPW,), jnp.int32),
            pltpu.VMEM((_PPW,), jnp.float32),
            pltpu.VMEM((_PPW,), jnp.float32),
            pltpu.SemaphoreType.DMA,
            pltpu.SemaphoreType.DMA,
        ],
    )


def _tc_body(starts_s, ends_s, starts_v, ends_v, w_v, px_v, py_v, out_ref, acc):
    i = pl.program_id(0)
    big = jnp.float32(3.0e38)
    lane = lax.broadcasted_iota(jnp.int32, (1, _CHUNK), 1)

    parts = []
    for s in range(_NB // _SUB):
        s_a = starts_s[0, 0, s * _SUB]
        s_b = ends_s[0, 0, s * _SUB + _SUB - 1]
        r0 = lax.div(s_a, _CHUNK)
        r1 = lax.div(s_b - 1, _CHUNK)
        nch = jnp.maximum(r1 - r0 + 1, 0)
        st = starts_v[0, s * _SUB:(s + 1) * _SUB].astype(jnp.int32)
        en = ends_v[0, s * _SUB:(s + 1) * _SUB].astype(jnp.int32)

        last = jnp.maximum(nch - 1, 0)

        def chunk(c, carry, st=st, en=en, r0=r0, last=last):
            cxmn, cxmx, cymn, cymx = carry
            for j in range(4):
                row = r0 + jnp.minimum(4 * c + j, last)
                pid = lane + row * _CHUNK
                pxc = px_v[pl.ds(row, 1), :]
                pyc = py_v[pl.ds(row, 1), :]
                member = (pid >= st) & (pid < en)
                cxmn = jnp.minimum(cxmn, jnp.where(member, pxc, big))
                cxmx = jnp.maximum(cxmx, jnp.where(member, pxc, -big))
                cymn = jnp.minimum(cymn, jnp.where(member, pyc, big))
                cymx = jnp.maximum(cymx, jnp.where(member, pyc, -big))
            return cxmn, cxmx, cymn, cymx

        cfull = jnp.full((_SUB, _CHUNK), big, jnp.float32)
        cxmn, cxmx, cymn, cymx = lax.fori_loop(
            0, lax.div(nch + 3, 4), chunk, (cfull, -cfull, cfull, -cfull))
        parts.append((jnp.min(cxmn, axis=1, keepdims=True),
                      jnp.max(cxmx, axis=1, keepdims=True),
                      jnp.min(cymn, axis=1, keepdims=True),
                      jnp.max(cymx, axis=1, keepdims=True)))

    xmn = jnp.concatenate([p[0] for p in parts], axis=0)
    xmx = jnp.concatenate([p[1] for p in parts], axis=0)
    ymn = jnp.concatenate([p[2] for p in parts], axis=0)
    ymx = jnp.concatenate([p[3] for p in parts], axis=0)
    st = starts_v[0].astype(jnp.int32)
    en = ends_v[0].astype(jnp.int32)

    valid = en > st
    xmn = jnp.where(valid, xmn, _XL)
    xmx = jnp.where(valid, xmx, _XL)
    ymn = jnp.where(valid, ymn, _YL)
    ymx = jnp.where(valid, ymx, _YL)
    wt = jnp.where(valid, w_v[0], 0.0)

    eps = jnp.float32(jnp.finfo(jnp.float32).eps)
    bxl = lax.broadcasted_iota(jnp.int32, (1, _NBX), 1).astype(jnp.float32) * _BSX + _XL
    byl = lax.broadcasted_iota(jnp.int32, (1, _NBY), 1).astype(jnp.float32) * _BSY + _YL
    ox = jnp.maximum(jnp.minimum(xmx, bxl + _BSX) - jnp.maximum(xmn, bxl), 0.0)
    oy = jnp.maximum(jnp.minimum(ymx, byl + _BSY) - jnp.maximum(ymn, byl), 0.0)
    wt_h = wt / (ymx - ymn + eps)
    wt_v = wt / (xmx - xmn + eps)
    a2 = jnp.concatenate([ox * wt_h, ox * wt_v], axis=1)

    contrib = lax.dot_general(a2, oy, (((0,), (0,)), ((), ())),
                              preferred_element_type=jnp.float32)

    @pl.when(i == 0)
    def _():
        acc[...] = jnp.zeros((2 * _NBX, _NBY), jnp.float32)

    acc[...] += contrib

    @pl.when(i == pl.num_programs(0) - 1)
    def _():
        bin_area = _BSX * _BSY
        sh = jnp.float32(1.0 / (bin_area * _UNIT_H_CAP))
        sv = jnp.float32(1.0 / (bin_area * _UNIT_V_CAP))
        out_ref[...] = jnp.maximum(jnp.abs(acc[0:_NBX, :]) * sh,
                                   jnp.abs(acc[_NBX:, :]) * sv)


def _clampi(i):
    return (jnp.minimum(i, _GRID - 1), 0, 0)


def _tc_rudy(starts3, ends3, startsv3, endsv3, w3, px2, py2):
    return pl.pallas_call(
        _tc_body,
        grid=(_GRID,),
        in_specs=[
            pl.BlockSpec((1, 1, _NB), _clampi, memory_space=pltpu.SMEM),
            pl.BlockSpec((1, 1, _NB), _clampi, memory_space=pltpu.SMEM),
            pl.BlockSpec((1, _NB, 1), _clampi),
            pl.BlockSpec((1, _NB, 1), _clampi),
            pl.BlockSpec((1, _NB, 1), _clampi),
            pl.BlockSpec((_PIN_ROWS_PAD, _CHUNK), lambda i: (0, 0)),
            pl.BlockSpec((_PIN_ROWS_PAD, _CHUNK), lambda i: (0, 0)),
        ],
        out_specs=pl.BlockSpec((_NBX, _NBY), lambda i: (0, 0)),
        out_shape=jax.ShapeDtypeStruct((_NBX, _NBY), jnp.float32),
        scratch_shapes=[pltpu.VMEM((2 * _NBX, _NBY), jnp.float32)],
    )(starts3, ends3, startsv3, endsv3, w3, px2, py2)


def kernel(pin_pos, netpin_start, flat_netpin, net_weights):
    x = pin_pos[:_NUM_PINS]
    y = pin_pos[_NUM_PINS:]
    fnp1 = jnp.pad(flat_netpin, (0, _PAD_PINS - _NUM_PINS))
    px1, py1 = _sc_gather()(x, y, fnp1)
    px2 = jnp.pad(px1.reshape(_PIN_ROWS, _CHUNK),
                  ((0, _PIN_ROWS_PAD - _PIN_ROWS), (0, 0)))
    py2 = jnp.pad(py1.reshape(_PIN_ROWS, _CHUNK),
                  ((0, _PIN_ROWS_PAD - _PIN_ROWS), (0, 0)))

    starts = jnp.pad(netpin_start[:-1], (0, _PAD_NETS - _NUM_NETS),
                     constant_values=_NUM_PINS)
    ends = jnp.pad(netpin_start[1:], (0, _PAD_NETS - _NUM_NETS),
                   constant_values=_NUM_PINS)
    wpad = jnp.pad(net_weights, (0, _PAD_NETS - _NUM_NETS))
    starts3 = starts.reshape(_GRID, 1, _NB)
    ends3 = ends.reshape(_GRID, 1, _NB)
    startsv3 = starts.reshape(_GRID, _NB, 1)
    endsv3 = ends.reshape(_GRID, _NB, 1)
    w3 = wpad.reshape(_GRID, _NB, 1)
    return _tc_rudy(starts3, ends3, startsv3, endsv3, w3, px2, py2)
